# baseline (speedup 1.0000x reference)
import functools

import jax
import jax.numpy as jnp
from jax import lax
from jax.experimental import pallas as pl
from jax.experimental.pallas import tpu as pltpu
from jax.experimental.pallas import tpu_sc as plsc

VOCAB = 1000
D = 1000
B = 4096
T = 20
NW = 32
NVT = D // 8
NBLK = T * NVT
BLK = 32 * 8 * 128
NPB = NBLK // NW

_mesh = plsc.VectorSubcoreMesh(core_axis_name="c", subcore_axis_name="s")


@functools.partial(
    pl.kernel,
    mesh=_mesh,
    out_type=jax.ShapeDtypeStruct((NBLK, BLK), jnp.float32),
    compiler_params=pltpu.CompilerParams(
        use_tc_tiling_on_sc=False, needs_layout_passes=False),
    scratch_types=[
        pltpu.VMEM((8, D), jnp.float32),
        pltpu.VMEM((B // 2,), jnp.int32),
        pltpu.VMEM((B // 2,), jnp.int32),
        pltpu.VMEM((B // 2,), jnp.int32),
        pltpu.VMEM((BLK,), jnp.float32),
        pltpu.VMEM((BLK,), jnp.float32),
        pltpu.VMEM((BLK,), jnp.float32),
        pltpu.SemaphoreType.DMA,
        pltpu.SemaphoreType.DMA,
        pltpu.SemaphoreType.DMA,
        pltpu.SemaphoreType.DMA,
        pltpu.SemaphoreType.DMA,
        pltpu.SemaphoreType.DMA,
    ],
)
def _gather_kernel(idxT_hbm, tableT_hbm, out_hbm,
                   rb, ibA, ibB, ibC, bbA, bbB, bbC, lA, lB, lC, wA, wB, wC):
    w = lax.axis_index("s") * 2 + lax.axis_index("c")
    start = NPB * w + jnp.minimum(w, 4)
    nblocks = NPB + (w < 4).astype(jnp.int32)

    ibs = (ibA, ibB, ibC)
    bbs = (bbA, bbB, bbC)
    lsems = (lA, lB, lC)
    wsems = (wA, wB, wC)
    NSLOT = 3

    def load_idx(nb, s):
        pltpu.async_copy(idxT_hbm.at[nb % T], ibs[s], lsems[s])

    def wait_idx(s):
        pltpu.make_async_copy(idxT_hbm.at[0], ibs[s], lsems[s]).wait()

    def write(t, vt, s):
        pltpu.async_copy(bbs[s], out_hbm.at[t * NVT + vt], wsems[s])

    def wait_write(s):
        pltpu.make_async_copy(bbs[s], out_hbm.at[0], wsems[s]).wait()

    def compute(s):
        ib, bb = ibs[s], bbs[s]

        @plsc.parallel_loop(0, 32, 1, unroll=2)
        def body_bt(bt):
            base = bt * 1024
            for m in range(4):
                iv = ib[pl.ds(bt * 64 + m * 16, 16)]
                lo = iv & jnp.int32(0xFFFF)
                hi = lax.shift_right_logical(iv, 16)
                for j, idx16 in ((2 * m, lo), (2 * m + 1, hi)):
                    vals = [plsc.load_gather(rb.at[vi], [idx16])
                            for vi in range(8)]
                    for vi in range(8):
                        bb[pl.ds(base + vi * 128 + j * 16, 16)] = vals[vi]

    load_idx(start, 0)
    load_idx(start + 1, 1)
    load_idx(start + 2, 2)

    def body(kk, carry):
        for s in range(NSLOT):
            k = NSLOT * kk + s
            nb = start + k

            @pl.when(k < nblocks)
            def _():
                vt = nb // T
                t = nb - vt * T

                @pl.when((t == 0) | (k == 0))
                def _():
                    pltpu.sync_copy(tableT_hbm.at[pl.ds(vt * 8, 8)], rb)

                wait_idx(s)

                @pl.when(kk > 0)
                def _():
                    wait_write(s)

                compute(s)
                write(t, vt, s)

                @pl.when(k + NSLOT < nblocks)
                def _():
                    load_idx(nb + NSLOT, s)

        return carry

    lax.fori_loop(0, (NPB + NSLOT) // NSLOT, body, 0)
    wait_write(0)
    wait_write(1)
    wait_write(2)


def kernel(idx, table):
    idxT = idx.T.reshape(T, 32, 4, 2, 16)
    idxP = (idxT[:, :, :, 0, :] | (idxT[:, :, :, 1, :] << 16)).reshape(T, B // 2)
    tableT = table.T
    out2 = _gather_kernel(idxP, tableT)
    out5 = out2.reshape(T, NVT, 32, 8, 128)
    return out5.transpose(2, 4, 0, 1, 3).reshape(B, T, D)


# speedup vs baseline: 1.0395x; 1.0395x over previous
import functools

import jax
import jax.numpy as jnp
from jax import lax
from jax.experimental import pallas as pl
from jax.experimental.pallas import tpu as pltpu
from jax.experimental.pallas import tpu_sc as plsc

VOCAB = 1000
D = 1000
B = 4096
T = 20
NW = 32
NVT = D // 8
NBLK = T * NVT
BLK = 32 * 8 * 128
NPB = NBLK // NW

_mesh = plsc.VectorSubcoreMesh(core_axis_name="c", subcore_axis_name="s")


@functools.partial(
    pl.kernel,
    mesh=_mesh,
    out_type=jax.ShapeDtypeStruct((NBLK, BLK), jnp.float32),
    compiler_params=pltpu.CompilerParams(
        use_tc_tiling_on_sc=False, needs_layout_passes=False),
    scratch_types=[
        pltpu.VMEM((8, D), jnp.float32),
        pltpu.VMEM((B // 2,), jnp.int32),
        pltpu.VMEM((B // 2,), jnp.int32),
        pltpu.VMEM((BLK,), jnp.float32),
        pltpu.VMEM((BLK,), jnp.float32),
        pltpu.SemaphoreType.DMA,
        pltpu.SemaphoreType.DMA,
        pltpu.SemaphoreType.DMA,
        pltpu.SemaphoreType.DMA,
    ],
)
def _gather_kernel(idxT_hbm, tableT_hbm, out_hbm,
                   rb, ibA, ibB, bbA, bbB, lA, lB, wA, wB):
    w = lax.axis_index("s") * 2 + lax.axis_index("c")
    start = NPB * w + jnp.minimum(w, 4)
    nblocks = NPB + (w < 4).astype(jnp.int32)

    ibs = (ibA, ibB)
    bbs = (bbA, bbB)
    lsems = (lA, lB)
    wsems = (wA, wB)
    NSLOT = 2

    def load_idx(nb, s):
        pltpu.async_copy(idxT_hbm.at[nb % T], ibs[s], lsems[s])

    def wait_idx(s):
        pltpu.make_async_copy(idxT_hbm.at[0], ibs[s], lsems[s]).wait()

    def write(t, vt, s):
        pltpu.async_copy(bbs[s], out_hbm.at[t * NVT + vt], wsems[s])

    def wait_write(s):
        pltpu.make_async_copy(bbs[s], out_hbm.at[0], wsems[s]).wait()

    def compute(s):
        ib, bb = ibs[s], bbs[s]

        @plsc.parallel_loop(0, 32, 1, unroll=2)
        def body_bt(bt):
            base = bt * 1024
            for m in range(4):
                iv = ib[pl.ds(bt * 64 + m * 16, 16)]
                lo = iv & jnp.int32(0xFFFF)
                hi = lax.shift_right_logical(iv, 16)
                for j, idx16 in ((2 * m, lo), (2 * m + 1, hi)):
                    vals = [plsc.load_gather(rb.at[vi], [idx16])
                            for vi in range(8)]
                    for vi in range(8):
                        bb[pl.ds(base + vi * 128 + j * 16, 16)] = vals[vi]

    load_idx(start, 0)
    load_idx(start + 1, 1)

    def body(kk, carry):
        for s in range(NSLOT):
            k = NSLOT * kk + s
            nb = start + k

            @pl.when(k < nblocks)
            def _():
                vt = nb // T
                t = nb - vt * T

                @pl.when((t == 0) | (k == 0))
                def _():
                    pltpu.sync_copy(tableT_hbm.at[pl.ds(vt * 8, 8)], rb)

                wait_idx(s)

                @pl.when(kk > 0)
                def _():
                    wait_write(s)

                compute(s)
                write(t, vt, s)

                @pl.when(k + NSLOT < nblocks)
                def _():
                    load_idx(nb + NSLOT, s)

        return carry

    lax.fori_loop(0, (NPB + NSLOT) // NSLOT, body, 0)
    wait_write(0)
    wait_write(1)


def kernel(idx, table):
    idxT = idx.T.reshape(T, 32, 4, 2, 16)
    idxP = (idxT[:, :, :, 0, :] | (idxT[:, :, :, 1, :] << 16)).reshape(T, B // 2)
    tableT = table.T
    out2 = _gather_kernel(idxP, tableT)
    out5 = out2.reshape(T, NVT, 32, 8, 128)
    return out5.transpose(2, 4, 0, 1, 3).reshape(B, T, D)


# speedup vs baseline: 1.1546x; 1.1106x over previous
import functools

import jax
import jax.numpy as jnp
from jax import lax
from jax.experimental import pallas as pl
from jax.experimental.pallas import tpu as pltpu
from jax.experimental.pallas import tpu_sc as plsc

VOCAB = 1000
D = 1000
B = 4096
T = 20
NW = 32
NVT = D // 8
NBLK = T * NVT
BLK = 32 * 8 * 128
NPB = NBLK // NW

_mesh = plsc.VectorSubcoreMesh(core_axis_name="c", subcore_axis_name="s")


@functools.partial(
    pl.kernel,
    mesh=_mesh,
    out_type=jax.ShapeDtypeStruct((NBLK, BLK), jnp.float32),
    compiler_params=pltpu.CompilerParams(
        use_tc_tiling_on_sc=False, needs_layout_passes=False),
    scratch_types=[
        pltpu.VMEM((8, D), jnp.float32),
        pltpu.VMEM((B // 2,), jnp.int32),
        pltpu.VMEM((B // 2,), jnp.int32),
        pltpu.VMEM((BLK,), jnp.float32),
        pltpu.VMEM((BLK,), jnp.float32),
        pltpu.SemaphoreType.DMA,
        pltpu.SemaphoreType.DMA,
        pltpu.SemaphoreType.DMA,
        pltpu.SemaphoreType.DMA,
    ],
)
def _gather_kernel(idxT_hbm, tableT_hbm, out_hbm,
                   rb, ibA, ibB, bbA, bbB, lA, lB, wA, wB):
    w = lax.axis_index("s") * 2 + lax.axis_index("c")
    start = NPB * w + jnp.minimum(w, 4)
    nblocks = NPB + (w < 4).astype(jnp.int32)

    ibs = (ibA, ibB)
    bbs = (bbA, bbB)
    lsems = (lA, lB)
    wsems = (wA, wB)
    NSLOT = 2

    def load_idx(nb, s):
        pltpu.async_copy(idxT_hbm.at[nb % T], ibs[s], lsems[s])

    def wait_idx(s):
        pltpu.make_async_copy(idxT_hbm.at[0], ibs[s], lsems[s]).wait()

    def write(t, vt, s):
        pltpu.async_copy(bbs[s], out_hbm.at[t * NVT + vt], wsems[s])

    def wait_write(s):
        pltpu.make_async_copy(bbs[s], out_hbm.at[0], wsems[s]).wait()

    def compute(s):
        ib, bb = ibs[s], bbs[s]

        @plsc.parallel_loop(0, 32, 1, unroll=4)
        def body_bt(bt):
            base = bt * 1024
            for m in range(4):
                iv = ib[pl.ds(bt * 64 + m * 16, 16)]
                lo = iv & jnp.int32(0xFFFF)
                hi = lax.shift_right_logical(iv, 16)
                for j, idx16 in ((2 * m, lo), (2 * m + 1, hi)):
                    vals = [plsc.load_gather(rb.at[vi], [idx16])
                            for vi in range(8)]
                    for vi in range(8):
                        bb[pl.ds(base + vi * 128 + j * 16, 16)] = vals[vi]

    load_idx(start, 0)
    load_idx(start + 1, 1)

    def body(kk, carry):
        for s in range(NSLOT):
            k = NSLOT * kk + s
            nb = start + k

            @pl.when(k < nblocks)
            def _():
                vt = nb // T
                t = nb - vt * T

                @pl.when((t == 0) | (k == 0))
                def _():
                    pltpu.sync_copy(tableT_hbm.at[pl.ds(vt * 8, 8)], rb)

                wait_idx(s)

                @pl.when(kk > 0)
                def _():
                    wait_write(s)

                compute(s)
                write(t, vt, s)

                @pl.when(k + NSLOT < nblocks)
                def _():
                    load_idx(nb + NSLOT, s)

        return carry

    lax.fori_loop(0, (NPB + NSLOT) // NSLOT, body, 0)
    wait_write(0)
    wait_write(1)


def kernel(idx, table):
    idxT = idx.T.reshape(T, 32, 4, 2, 16)
    idxP = (idxT[:, :, :, 0, :] | (idxT[:, :, :, 1, :] << 16)).reshape(T, B // 2)
    tableT = table.T
    out2 = _gather_kernel(idxP, tableT)
    out5 = out2.reshape(T, NVT, 32, 8, 128)
    return out5.transpose(2, 4, 0, 1, 3).reshape(B, T, D)
